# baseline (speedup 1.0000x reference)
import functools

import jax
import jax.numpy as jnp
from jax import lax
from jax.experimental import pallas as pl
from jax.experimental.pallas import tpu as pltpu
from jax.experimental.pallas import tpu_sc as plsc

_L = 16

_SC_PARAMS = pltpu.CompilerParams(needs_layout_passes=False)


def _mesh():
    return plsc.VectorSubcoreMesh(core_axis_name="c", subcore_axis_name="s")


@functools.lru_cache(maxsize=None)
def _make_dispatch(NK, E, C, K, H):
    assert E == _L and NK % _L == 0
    N = NK // K
    assert N & (N - 1) == 0
    SEG = NK // _L
    EC = E * C
    H2 = H // 2
    mesh = _mesh()
    NW = mesh.num_cores * mesh.num_subcores
    RP = N // NW
    RCH = 8
    assert RP % RCH == 0

    @functools.partial(
        pl.kernel, mesh=mesh,
        out_type=[jax.ShapeDtypeStruct((EC,), jnp.int32),
                  jax.ShapeDtypeStruct((NK,), jnp.int32),
                  jax.ShapeDtypeStruct((NK,), jnp.float32),
                  jax.ShapeDtypeStruct((N, H2), jnp.int32)],
        scratch_types=[pltpu.VMEM((NK,), jnp.int32),
                       pltpu.VMEM((NK,), jnp.float32),
                       pltpu.VMEM((EC,), jnp.int32),
                       pltpu.VMEM((NK,), jnp.int32),
                       pltpu.VMEM((NK,), jnp.float32),
                       pltpu.VMEM((_L * E,), jnp.int32),
                       pltpu.VMEM((_L * E,), jnp.int32),
                       pltpu.VMEM((RCH, H), jnp.float32),
                       pltpu.VMEM((RCH, H2), jnp.int32)],
        compiler_params=_SC_PARAMS,
    )
    def dispatch(ids_hbm, w_hbm, hid_hbm, tok_hbm, inv_hbm, wv_hbm, xpk_hbm,
                 ids_v, w_v, tok_v, inv_v, wv_v, ctab, htab, xin_v, xout_v):
        cid = lax.axis_index("c")
        sid = lax.axis_index("s")
        is0 = (cid == 0) & (sid == 0)
        wid = sid * mesh.num_cores + cid

        r0 = wid * RP
        for chunk in range(RP // RCH):
            rb = r0 + chunk * RCH
            pltpu.sync_copy(hid_hbm.at[pl.ds(rb, RCH)], xin_v)

            def pk(j, c, _chunk=chunk):
                for r in range(RCH):
                    va = xin_v[r, pl.ds(j * _L, _L)]
                    vb = xin_v[r, pl.ds(H2 + j * _L, _L)]
                    pr = plsc.pack(va, vb, format=plsc.PackFormat.INTERLEAVED)
                    xout_v[r, pl.ds(j * _L, _L)] = plsc.bitcast(pr, jnp.int32)
                return c
            lax.fori_loop(0, H2 // _L, pk, 0)
            pltpu.sync_copy(xout_v, xpk_hbm.at[pl.ds(rb, RCH)])

        @pl.when(is0)
        def _():
            pltpu.sync_copy(ids_hbm, ids_v)
            pltpu.sync_copy(w_hbm, w_v)
            lanes = lax.iota(jnp.int32, _L)
            zeros16 = jnp.zeros((_L,), jnp.int32)
            ones16 = jnp.ones((_L,), jnp.int32)
            for b in range(_L):
                htab[pl.ds(b * E, E)] = zeros16

            def zb(i, c):
                tok_v[pl.ds(i * _L, _L)] = (
                    (i * jnp.int32(_L) + lanes) & jnp.int32(N - 1))
                return c
            lax.fori_loop(0, EC // _L, zb, 0)

            lane_base = lanes * jnp.int32(SEG)
            tab_base = lanes * jnp.int32(E)

            def p1(t, c):
                v = plsc.load_gather(ids_v, [lane_base + t])
                plsc.addupdate_scatter(htab, [tab_base + v], ones16)
                return c
            lax.fori_loop(0, SEG, p1, 0)

            run = zeros16
            for l in range(_L):
                ctab[pl.ds(l * E, E)] = run
                run = run + htab[pl.ds(l * E, E)]

            Cc = jnp.int32(C)

            def p2(t, c):
                pos = lane_base + t
                v = plsc.load_gather(ids_v, [pos])
                li = tab_base + v
                rank = plsc.load_gather(ctab, [li])
                plsc.addupdate_scatter(ctab, [li], ones16)
                valid = rank < Cc
                slot = v * Cc + rank
                tok = pos // jnp.int32(K)
                plsc.store_scatter(tok_v, [slot], tok, mask=valid)
                plsc.store_scatter(inv_v, [pos], jnp.where(valid, slot, 0))
                wval = plsc.load_gather(w_v, [pos])
                wval = jnp.where(valid, wval, jnp.zeros((_L,), jnp.float32))
                plsc.store_scatter(wv_v, [pos], wval)
                return c
            lax.fori_loop(0, SEG, p2, 0)

            pltpu.sync_copy(tok_v, tok_hbm)
            pltpu.sync_copy(inv_v, inv_hbm)
            pltpu.sync_copy(wv_v, wv_hbm)

    return dispatch


@functools.lru_cache(maxsize=None)
def _make_cast(N, H):
    BN = 256
    assert N % BN == 0

    def body(x_ref, o_ref):
        o_ref[...] = x_ref[...].astype(jnp.bfloat16)

    return pl.pallas_call(
        body,
        grid=(N // BN,),
        in_specs=[pl.BlockSpec((BN, H), lambda i: (i, 0))],
        out_specs=pl.BlockSpec((BN, H), lambda i: (i, 0)),
        out_shape=jax.ShapeDtypeStruct((N, H), jnp.bfloat16),
    )


@functools.lru_cache(maxsize=None)
def _make_gather(EC, N, H, Q0=8):
    mesh = _mesh()
    NS = mesh.num_subcores
    CH = 16
    assert EC % (16 * NS * CH) == 0
    R0 = EC * Q0 // 16
    RPW0 = R0 // NS
    RPW1 = (EC - R0) // NS
    NCH0 = RPW0 // CH
    NCH1 = RPW1 // CH
    NB = 3

    @functools.partial(
        pl.kernel, mesh=mesh,
        out_type=jax.ShapeDtypeStruct((EC, H), jnp.int32),
        scratch_types=[pltpu.VMEM((max(RPW0, RPW1),), jnp.int32),
                       pltpu.VMEM((NB, CH, H), jnp.int32),
                       pltpu.SemaphoreType.DMA,
                       pltpu.SemaphoreType.DMA,
                       pltpu.SemaphoreType.DMA,
                       pltpu.SemaphoreType.DMA,
                       pltpu.SemaphoreType.DMA],
        compiler_params=_SC_PARAMS,
    )
    def gather(tok_hbm, hid_hbm, xg_hbm, idx_v, buf,
               gs0, gs1, os0, os1, os2):
        cid = lax.axis_index("c")
        sid = lax.axis_index("s")
        gsems = (gs0, gs1)
        osems = (os0, os1, os2)

        def run(base, rpw, nch):
            pltpu.sync_copy(tok_hbm.at[pl.ds(base, rpw)],
                            idx_v.at[pl.ds(0, rpw)])

            def g_start(c):
                return pltpu.async_copy(
                    hid_hbm.at[idx_v.at[pl.ds(c * CH, CH)]],
                    buf.at[c % NB], gsems[c % 2])

            def o_start(c):
                return pltpu.async_copy(
                    buf.at[c % NB],
                    xg_hbm.at[pl.ds(base + c * CH, CH)], osems[c % 3])

            g = {0: g_start(0)}
            if nch > 1:
                g[1] = g_start(1)
            o = {}
            for c in range(nch):
                g[c].wait()
                o[c] = o_start(c)
                nxt = c + 2
                if nxt < nch:
                    if nxt - NB >= 0:
                        o[nxt - NB].wait()
                    g[nxt] = g_start(nxt)
            for c in range(max(0, nch - NB), nch):
                o[c].wait()

        if NCH0 > 0:
            @pl.when(cid == 0)
            def _():
                run(sid * RPW0, RPW0, NCH0)

        @pl.when(cid == 1)
        def _():
            run(R0 + sid * RPW1, RPW1, NCH1)

    return gather


@functools.lru_cache(maxsize=None)
def _make_ffn_up(E, C, H, I):
    NJ = 4
    BI = I // NJ
    H2 = H // 2

    def body(x_ref, wg_ref, wu_ref, at_ref, xs_ref):
        jb = pl.program_id(1)

        @pl.when(jb == 0)
        def _():
            xi = x_ref[0]
            lo = lax.bitcast_convert_type(
                lax.shift_left(xi, jnp.int32(16)), jnp.float32)
            hi = lax.bitcast_convert_type(
                lax.bitwise_and(xi, jnp.int32(-65536)), jnp.float32)
            xs_ref[:, :H2] = lo.astype(jnp.bfloat16)
            xs_ref[:, H2:] = hi.astype(jnp.bfloat16)

        x = xs_ref[...]
        gt = lax.dot_general(wg_ref[0].astype(jnp.bfloat16), x,
                             (((1,), (1,)), ((), ())),
                             preferred_element_type=jnp.float32)
        ut = lax.dot_general(wu_ref[0].astype(jnp.bfloat16), x,
                             (((1,), (1,)), ((), ())),
                             preferred_element_type=jnp.float32)
        at = gt * jax.nn.sigmoid(gt) * ut
        at_ref[...] = at.astype(jnp.bfloat16)[None]

    return pl.pallas_call(
        body,
        grid=(E, NJ),
        in_specs=[
            pl.BlockSpec((1, C, H2), lambda e, j: (e, 0, 0)),
            pl.BlockSpec((1, BI, H), lambda e, j: (e, j, 0)),
            pl.BlockSpec((1, BI, H), lambda e, j: (e, NJ + j, 0)),
        ],
        out_specs=pl.BlockSpec((1, BI, C), lambda e, j: (e, j, 0)),
        out_shape=jax.ShapeDtypeStruct((E, I, C), jnp.bfloat16),
        scratch_shapes=[pltpu.VMEM((C, H), jnp.bfloat16)],
        compiler_params=pltpu.CompilerParams(
            dimension_semantics=("parallel", "arbitrary"),
            vmem_limit_bytes=120 * 1024 * 1024),
    )


@functools.lru_cache(maxsize=None)
def _make_ffn_down(E, C, H, I):
    def body(at_ref, w2_ref, y_ref):
        yb = lax.dot_general(at_ref[0], w2_ref[0].astype(jnp.bfloat16),
                             (((0,), (1,)), ((), ())),
                             preferred_element_type=jnp.float32)
        y_ref[...] = yb[None]

    return pl.pallas_call(
        body,
        grid=(E,),
        in_specs=[
            pl.BlockSpec((1, I, C), lambda e: (e, 0, 0)),
            pl.BlockSpec((1, H, I), lambda e: (e, 0, 0)),
        ],
        out_specs=pl.BlockSpec((1, C, H), lambda e: (e, 0, 0)),
        out_shape=jax.ShapeDtypeStruct((E, C, H), jnp.float32),
        compiler_params=pltpu.CompilerParams(
            dimension_semantics=("arbitrary",),
            vmem_limit_bytes=120 * 1024 * 1024),
    )


@functools.lru_cache(maxsize=None)
def _make_combine(EC, N, K, H):
    mesh = _mesh()
    NW = mesh.num_cores * mesh.num_subcores
    assert N % NW == 0 and H % _L == 0
    TPW = N // NW
    CT = 8
    assert TPW % CT == 0
    NCH = TPW // CT
    G = H // _L

    @functools.partial(
        pl.kernel, mesh=mesh,
        out_type=jax.ShapeDtypeStruct((N, H), jnp.float32),
        scratch_types=[pltpu.VMEM((TPW * K,), jnp.int32),
                       pltpu.VMEM((TPW * K,), jnp.float32),
                       pltpu.VMEM((2, CT * K, H), jnp.float32),
                       pltpu.VMEM((CT, H), jnp.float32),
                       pltpu.SemaphoreType.DMA,
                       pltpu.SemaphoreType.DMA],
        compiler_params=_SC_PARAMS,
    )
    def combine(y_hbm, inv_hbm, wv_hbm, out_hbm,
                idx_v, w_v, rows, out_v, sem0, sem1):
        nc = mesh.num_cores
        wid = lax.axis_index("s") * nc + lax.axis_index("c")
        tbase = wid * TPW
        pltpu.sync_copy(inv_hbm.at[pl.ds(tbase * K, TPW * K)], idx_v)
        pltpu.sync_copy(wv_hbm.at[pl.ds(tbase * K, TPW * K)], w_v)
        sems = (sem0, sem1)

        def start(cc):
            return pltpu.async_copy(
                y_hbm.at[idx_v.at[pl.ds(cc * CT * K, CT * K)]],
                rows.at[cc % 2], sems[cc % 2])

        handles = {0: start(0)}
        for cc in range(NCH):
            if cc + 1 < NCH:
                handles[cc + 1] = start(cc + 1)
            handles[cc].wait()
            b = cc % 2
            wvec = w_v[pl.ds(cc * CT * K, CT * K)]

            def gbody(g, c, cc=cc, b=b, wvec=wvec):
                hs = g * jnp.int32(_L)
                for i in range(CT):
                    acc = None
                    for k in range(K):
                        w = wvec[i * K + k]
                        r = rows[b, i * K + k, pl.ds(hs, _L)]
                        term = r * w
                        acc = term if acc is None else acc + term
                    out_v[i, pl.ds(hs, _L)] = acc
                return c
            lax.fori_loop(0, G, gbody, 0)
            pltpu.sync_copy(out_v, out_hbm.at[pl.ds(tbase + cc * CT, CT)])

    return combine


def kernel(hidden_states, topk_ids, topk_weights, router_logits,
           w13_weight, w2_weight):
    del router_logits
    N, H = hidden_states.shape
    K = topk_ids.shape[1]
    E = w13_weight.shape[0]
    I = w2_weight.shape[2]
    NK = N * K
    C = max(1, (2 * NK) // E)
    EC = E * C

    ids = topk_ids.reshape(-1).astype(jnp.int32)
    wts = topk_weights.reshape(-1).astype(jnp.float32)
    x = hidden_states.astype(jnp.float32)

    tok_buf, inv_slot, wv, x_pk = _make_dispatch(NK, E, C, K, H)(ids, wts, x)
    xg = _make_gather(EC, N, H // 2, 8)(tok_buf, x_pk)
    at = _make_ffn_up(E, C, H, I)(
        xg.reshape(E, C, H // 2), w13_weight, w13_weight)
    y = _make_ffn_down(E, C, H, I)(at, w2_weight)
    out = _make_combine(EC, N, K, H)(y.reshape(EC, H), inv_slot, wv)
    return out


# speedup vs baseline: 1.0533x; 1.0533x over previous
import functools

import jax
import jax.numpy as jnp
from jax import lax
from jax.experimental import pallas as pl
from jax.experimental.pallas import tpu as pltpu
from jax.experimental.pallas import tpu_sc as plsc

_L = 16

_SC_PARAMS = pltpu.CompilerParams(needs_layout_passes=False)


def _mesh():
    return plsc.VectorSubcoreMesh(core_axis_name="c", subcore_axis_name="s")


@functools.lru_cache(maxsize=None)
def _make_dispatch(NK, E, C, K, H):
    assert E == _L and NK % _L == 0
    N = NK // K
    assert N & (N - 1) == 0
    SEG = NK // _L
    EC = E * C
    H2 = H // 2
    mesh = _mesh()
    NW = mesh.num_cores * mesh.num_subcores
    RP = N // NW
    RCH = 8
    assert RP % RCH == 0

    @functools.partial(
        pl.kernel, mesh=mesh,
        out_type=[jax.ShapeDtypeStruct((EC,), jnp.int32),
                  jax.ShapeDtypeStruct((NK,), jnp.int32),
                  jax.ShapeDtypeStruct((NK,), jnp.float32),
                  jax.ShapeDtypeStruct((N, H2), jnp.int32)],
        scratch_types=[pltpu.VMEM((NK,), jnp.int32),
                       pltpu.VMEM((NK,), jnp.float32),
                       pltpu.VMEM((EC,), jnp.int32),
                       pltpu.VMEM((NK,), jnp.int32),
                       pltpu.VMEM((NK,), jnp.float32),
                       pltpu.VMEM((_L * E,), jnp.int32),
                       pltpu.VMEM((_L * E,), jnp.int32),
                       pltpu.VMEM((RCH, H), jnp.float32),
                       pltpu.VMEM((RCH, H2), jnp.int32)],
        compiler_params=_SC_PARAMS,
    )
    def dispatch(ids_hbm, w_hbm, hid_hbm, tok_hbm, inv_hbm, wv_hbm, xpk_hbm,
                 ids_v, w_v, tok_v, inv_v, wv_v, ctab, htab, xin_v, xout_v):
        cid = lax.axis_index("c")
        sid = lax.axis_index("s")
        is0 = (cid == 0) & (sid == 0)
        wid = sid * mesh.num_cores + cid

        r0 = wid * RP
        for chunk in range(RP // RCH):
            rb = r0 + chunk * RCH
            pltpu.sync_copy(hid_hbm.at[pl.ds(rb, RCH)], xin_v)

            def pk(j, c, _chunk=chunk):
                for r in range(RCH):
                    va = xin_v[r, pl.ds(j * _L, _L)]
                    vb = xin_v[r, pl.ds(H2 + j * _L, _L)]
                    pr = plsc.pack(va, vb, format=plsc.PackFormat.INTERLEAVED)
                    xout_v[r, pl.ds(j * _L, _L)] = plsc.bitcast(pr, jnp.int32)
                return c
            lax.fori_loop(0, H2 // _L, pk, 0)
            pltpu.sync_copy(xout_v, xpk_hbm.at[pl.ds(rb, RCH)])

        @pl.when(is0)
        def _():
            pltpu.sync_copy(ids_hbm, ids_v)
            pltpu.sync_copy(w_hbm, w_v)
            lanes = lax.iota(jnp.int32, _L)
            zeros16 = jnp.zeros((_L,), jnp.int32)
            ones16 = jnp.ones((_L,), jnp.int32)
            for b in range(_L):
                htab[pl.ds(b * E, E)] = zeros16

            def zb(i, c):
                tok_v[pl.ds(i * _L, _L)] = (
                    (i * jnp.int32(_L) + lanes) & jnp.int32(N - 1))
                return c
            lax.fori_loop(0, EC // _L, zb, 0)

            lane_base = lanes * jnp.int32(SEG)
            tab_base = lanes * jnp.int32(E)

            def p1(t, c):
                v = plsc.load_gather(ids_v, [lane_base + t])
                plsc.addupdate_scatter(htab, [tab_base + v], ones16)
                return c
            lax.fori_loop(0, SEG, p1, 0)

            run = zeros16
            for l in range(_L):
                ctab[pl.ds(l * E, E)] = run
                run = run + htab[pl.ds(l * E, E)]

            Cc = jnp.int32(C)

            def p2(t, c):
                pos = lane_base + t
                v = plsc.load_gather(ids_v, [pos])
                li = tab_base + v
                rank = plsc.load_gather(ctab, [li])
                plsc.addupdate_scatter(ctab, [li], ones16)
                valid = rank < Cc
                slot = v * Cc + rank
                tok = pos // jnp.int32(K)
                plsc.store_scatter(tok_v, [slot], tok, mask=valid)
                plsc.store_scatter(inv_v, [pos], jnp.where(valid, slot, 0))
                wval = plsc.load_gather(w_v, [pos])
                wval = jnp.where(valid, wval, jnp.zeros((_L,), jnp.float32))
                plsc.store_scatter(wv_v, [pos], wval)
                return c
            lax.fori_loop(0, SEG, p2, 0)

            pltpu.sync_copy(tok_v, tok_hbm)
            pltpu.sync_copy(inv_v, inv_hbm)
            pltpu.sync_copy(wv_v, wv_hbm)

    return dispatch


@functools.lru_cache(maxsize=None)
def _make_cast(N, H):
    BN = 256
    assert N % BN == 0

    def body(x_ref, o_ref):
        o_ref[...] = x_ref[...].astype(jnp.bfloat16)

    return pl.pallas_call(
        body,
        grid=(N // BN,),
        in_specs=[pl.BlockSpec((BN, H), lambda i: (i, 0))],
        out_specs=pl.BlockSpec((BN, H), lambda i: (i, 0)),
        out_shape=jax.ShapeDtypeStruct((N, H), jnp.bfloat16),
    )


@functools.lru_cache(maxsize=None)
def _make_gather(EC, N, H, Q0=8):
    mesh = _mesh()
    NS = mesh.num_subcores
    CH = 16
    assert EC % (16 * NS * CH) == 0
    R0 = EC * Q0 // 16
    RPW0 = R0 // NS
    RPW1 = (EC - R0) // NS
    NCH0 = RPW0 // CH
    NCH1 = RPW1 // CH
    NB = 3

    @functools.partial(
        pl.kernel, mesh=mesh,
        out_type=jax.ShapeDtypeStruct((EC, H), jnp.int32),
        scratch_types=[pltpu.VMEM((max(RPW0, RPW1),), jnp.int32),
                       pltpu.VMEM((NB, CH, H), jnp.int32),
                       pltpu.SemaphoreType.DMA,
                       pltpu.SemaphoreType.DMA,
                       pltpu.SemaphoreType.DMA,
                       pltpu.SemaphoreType.DMA,
                       pltpu.SemaphoreType.DMA],
        compiler_params=_SC_PARAMS,
    )
    def gather(tok_hbm, hid_hbm, xg_hbm, idx_v, buf,
               gs0, gs1, os0, os1, os2):
        cid = lax.axis_index("c")
        sid = lax.axis_index("s")
        gsems = (gs0, gs1)
        osems = (os0, os1, os2)

        def run(base, rpw, nch):
            pltpu.sync_copy(tok_hbm.at[pl.ds(base, rpw)],
                            idx_v.at[pl.ds(0, rpw)])

            def g_start(c):
                return pltpu.async_copy(
                    hid_hbm.at[idx_v.at[pl.ds(c * CH, CH)]],
                    buf.at[c % NB], gsems[c % 2])

            def o_start(c):
                return pltpu.async_copy(
                    buf.at[c % NB],
                    xg_hbm.at[pl.ds(base + c * CH, CH)], osems[c % 3])

            g = {0: g_start(0)}
            if nch > 1:
                g[1] = g_start(1)
            o = {}
            for c in range(nch):
                g[c].wait()
                o[c] = o_start(c)
                nxt = c + 2
                if nxt < nch:
                    if nxt - NB >= 0:
                        o[nxt - NB].wait()
                    g[nxt] = g_start(nxt)
            for c in range(max(0, nch - NB), nch):
                o[c].wait()

        if NCH0 > 0:
            @pl.when(cid == 0)
            def _():
                run(sid * RPW0, RPW0, NCH0)

        @pl.when(cid == 1)
        def _():
            run(R0 + sid * RPW1, RPW1, NCH1)

    return gather


@functools.lru_cache(maxsize=None)
def _make_ffn_up(E, C, H, I):
    NJ = 2
    BI = I // NJ
    H2 = H // 2

    def body(x_ref, wg_ref, wu_ref, at_ref, xs_ref):
        jb = pl.program_id(1)

        @pl.when(jb == 0)
        def _():
            xi = x_ref[0]
            lo = lax.bitcast_convert_type(
                lax.shift_left(xi, jnp.int32(16)), jnp.float32)
            hi = lax.bitcast_convert_type(
                lax.bitwise_and(xi, jnp.int32(-65536)), jnp.float32)
            xs_ref[:, :H2] = lo.astype(jnp.bfloat16)
            xs_ref[:, H2:] = hi.astype(jnp.bfloat16)

        x = xs_ref[...]
        gt = lax.dot_general(wg_ref[0].astype(jnp.bfloat16), x,
                             (((1,), (1,)), ((), ())),
                             preferred_element_type=jnp.float32)
        ut = lax.dot_general(wu_ref[0].astype(jnp.bfloat16), x,
                             (((1,), (1,)), ((), ())),
                             preferred_element_type=jnp.float32)
        at = gt * jax.nn.sigmoid(gt) * ut
        at_ref[...] = at.astype(jnp.bfloat16)[None]

    return pl.pallas_call(
        body,
        grid=(E, NJ),
        in_specs=[
            pl.BlockSpec((1, C, H2), lambda e, j: (e, 0, 0)),
            pl.BlockSpec((1, BI, H), lambda e, j: (e, j, 0)),
            pl.BlockSpec((1, BI, H), lambda e, j: (e, NJ + j, 0)),
        ],
        out_specs=pl.BlockSpec((1, BI, C), lambda e, j: (e, j, 0)),
        out_shape=jax.ShapeDtypeStruct((E, I, C), jnp.bfloat16),
        scratch_shapes=[pltpu.VMEM((C, H), jnp.bfloat16)],
        compiler_params=pltpu.CompilerParams(
            dimension_semantics=("parallel", "arbitrary"),
            vmem_limit_bytes=120 * 1024 * 1024),
    )


@functools.lru_cache(maxsize=None)
def _make_ffn_down(E, C, H, I):
    def body(at_ref, w2_ref, y_ref):
        yb = lax.dot_general(at_ref[0], w2_ref[0].astype(jnp.bfloat16),
                             (((0,), (1,)), ((), ())),
                             preferred_element_type=jnp.float32)
        y_ref[...] = yb[None]

    return pl.pallas_call(
        body,
        grid=(E,),
        in_specs=[
            pl.BlockSpec((1, I, C), lambda e: (e, 0, 0)),
            pl.BlockSpec((1, H, I), lambda e: (e, 0, 0)),
        ],
        out_specs=pl.BlockSpec((1, C, H), lambda e: (e, 0, 0)),
        out_shape=jax.ShapeDtypeStruct((E, C, H), jnp.float32),
        compiler_params=pltpu.CompilerParams(
            dimension_semantics=("arbitrary",),
            vmem_limit_bytes=120 * 1024 * 1024),
    )


@functools.lru_cache(maxsize=None)
def _make_combine(EC, N, K, H):
    mesh = _mesh()
    NW = mesh.num_cores * mesh.num_subcores
    assert N % NW == 0 and H % _L == 0
    TPW = N // NW
    CT = 8
    assert TPW % CT == 0
    NCH = TPW // CT
    G = H // _L

    @functools.partial(
        pl.kernel, mesh=mesh,
        out_type=jax.ShapeDtypeStruct((N, H), jnp.float32),
        scratch_types=[pltpu.VMEM((TPW * K,), jnp.int32),
                       pltpu.VMEM((TPW * K,), jnp.float32),
                       pltpu.VMEM((2, CT * K, H), jnp.float32),
                       pltpu.VMEM((CT, H), jnp.float32),
                       pltpu.SemaphoreType.DMA,
                       pltpu.SemaphoreType.DMA],
        compiler_params=_SC_PARAMS,
    )
    def combine(y_hbm, inv_hbm, wv_hbm, out_hbm,
                idx_v, w_v, rows, out_v, sem0, sem1):
        nc = mesh.num_cores
        wid = lax.axis_index("s") * nc + lax.axis_index("c")
        tbase = wid * TPW
        pltpu.sync_copy(inv_hbm.at[pl.ds(tbase * K, TPW * K)], idx_v)
        pltpu.sync_copy(wv_hbm.at[pl.ds(tbase * K, TPW * K)], w_v)
        sems = (sem0, sem1)

        def start(cc):
            return pltpu.async_copy(
                y_hbm.at[idx_v.at[pl.ds(cc * CT * K, CT * K)]],
                rows.at[cc % 2], sems[cc % 2])

        handles = {0: start(0)}
        for cc in range(NCH):
            if cc + 1 < NCH:
                handles[cc + 1] = start(cc + 1)
            handles[cc].wait()
            b = cc % 2
            wvec = w_v[pl.ds(cc * CT * K, CT * K)]

            def gbody(g, c, cc=cc, b=b, wvec=wvec):
                hs = g * jnp.int32(_L)
                for i in range(CT):
                    acc = None
                    for k in range(K):
                        w = wvec[i * K + k]
                        r = rows[b, i * K + k, pl.ds(hs, _L)]
                        term = r * w
                        acc = term if acc is None else acc + term
                    out_v[i, pl.ds(hs, _L)] = acc
                return c
            lax.fori_loop(0, G, gbody, 0)
            pltpu.sync_copy(out_v, out_hbm.at[pl.ds(tbase + cc * CT, CT)])

    return combine


def kernel(hidden_states, topk_ids, topk_weights, router_logits,
           w13_weight, w2_weight):
    del router_logits
    N, H = hidden_states.shape
    K = topk_ids.shape[1]
    E = w13_weight.shape[0]
    I = w2_weight.shape[2]
    NK = N * K
    C = max(1, (2 * NK) // E)
    EC = E * C

    ids = topk_ids.reshape(-1).astype(jnp.int32)
    wts = topk_weights.reshape(-1).astype(jnp.float32)
    x = hidden_states.astype(jnp.float32)

    tok_buf, inv_slot, wv, x_pk = _make_dispatch(NK, E, C, K, H)(ids, wts, x)
    xg = _make_gather(EC, N, H // 2, 8)(tok_buf, x_pk)
    at = _make_ffn_up(E, C, H, I)(
        xg.reshape(E, C, H // 2), w13_weight, w13_weight)
    y = _make_ffn_down(E, C, H, I)(at, w2_weight)
    out = _make_combine(EC, N, K, H)(y.reshape(EC, H), inv_slot, wv)
    return out
